# baseline (speedup 1.0000x reference)
import jax
import jax.numpy as jnp
from jax import lax
from jax.experimental import pallas as pl
from jax.experimental.pallas import tpu as pltpu
from jax.experimental.pallas import tpu_sc as plsc

_DIM = 64
_NJ = 20
_NI = 16384
_NC = 2
_NS = 16
_NW = _NC * _NS

_IPW = _NI // _NW
_P = 128
_NPAN = _NJ * (_IPW // _P)
_HALF = 500000
_RB = 80
_NRB = 2 * _HALF // _RB


def _repack_body(w_hbm, w2_hbm, bufa_v, bufb_v, bufc_v,
                 sa0, sa1, sb0, sb1, sc0, sc1):
    wid = lax.axis_index("s") * _NC + lax.axis_index("c")
    sem_a = (sa0, sa1)
    sem_b = (sb0, sb1)
    sem_c = (sc0, sc1)

    def start_in(bb, b):
        r = bb * _RB
        pltpu.async_copy(w_hbm.at[pl.ds(r, _RB), :], bufa_v.at[b], sem_a[b])
        pltpu.async_copy(
            w_hbm.at[pl.ds(_HALF + r, _RB), :], bufb_v.at[b], sem_b[b])

    def wait_in(b):
        pltpu.make_async_copy(w_hbm.at[pl.ds(0, _RB), :], bufa_v.at[b],
                              sem_a[b]).wait()
        pltpu.make_async_copy(w_hbm.at[pl.ds(0, _RB), :], bufb_v.at[b],
                              sem_b[b]).wait()

    def out_start(bb, b):
        pltpu.async_copy(bufc_v.at[b], w2_hbm.at[pl.ds(bb * _RB, _RB)],
                         sem_c[b])

    def out_wait(b):
        pltpu.make_async_copy(bufc_v.at[b], w2_hbm.at[pl.ds(0, _RB)],
                              sem_c[b]).wait()

    def interleave(b):
        def per_r(r, _):
            for q in range(_DIM // 16):
                bufc_v[b, r, pl.ds(16 * q, 16)] = bufa_v[b, r, pl.ds(16 * q, 16)]
                bufc_v[b, r, pl.ds(_DIM + 16 * q, 16)] = (
                    bufb_v[b, r, pl.ds(16 * q, 16)])
            return ()

        lax.fori_loop(0, _RB, per_r, (), unroll=4)

    nsteps = (_HALF // _RB + 2 * _NW - 1) // (2 * _NW)

    def step(i, _):
        for b in range(2):
            bb = (2 * i + b) * _NW + wid

            @pl.when(bb < _HALF // _RB)
            def _():
                @pl.when(i > 0)
                def _():
                    out_wait(b)

                start_in(bb, b)
                wait_in(b)
                interleave(b)
                out_start(bb, b)
        return ()

    lax.fori_loop(0, nsteps, step, (), unroll=False)
    for b in range(2):
        out_wait(b)


def _emb_body(xT_hbm, w2_hbm, outT_hbm, xT_v, idx_v, hsel_v, rows_v, pan_v,
              sg0, sg1, sg2, sg3, so0, so1):
    sem_g = (sg0, sg1, sg2, sg3)
    sem_o = (so0, so1)
    wid = lax.axis_index("s") * _NC + lax.axis_index("c")
    i0 = wid * _IPW

    pltpu.sync_copy(xT_hbm.at[:, pl.ds(i0, _IPW)], xT_v)

    def prep(g, _):
        j = g // (_IPW // 16)
        t = g % (_IPW // 16)
        v = xT_v[j, pl.ds(t * 16, 16)]
        hi = jnp.where(v >= _HALF, 1, 0)
        idx_v[pl.ds(g * 16, 16)] = v - hi * _HALF
        hsel_v[pl.ds(g * 16, 16)] = lax.shift_left(hi, 6)
        return ()

    lax.fori_loop(0, _NJ * (_IPW // 16), prep, (), unroll=False)

    def gather(pp, b):
        pltpu.async_copy(
            w2_hbm.at[idx_v.at[pl.ds(pp * _P, _P)]], rows_v.at[b], sem_g[b])

    def gather_wait(pp, b):
        pltpu.make_async_copy(
            w2_hbm.at[idx_v.at[pl.ds(pp * _P, _P)]], rows_v.at[b],
            sem_g[b]).wait()

    def pan_dst(pp, b):
        j = pp // (_IPW // _P)
        blk = pp % (_IPW // _P)
        return outT_hbm.at[j, :, pl.ds(i0 + blk * _P, _P)]

    def pan_start(pp, b):
        pltpu.async_copy(pan_v.at[b], pan_dst(pp, b), sem_o[b])

    def pan_wait(pp, b):
        pltpu.make_async_copy(pan_v.at[b], pan_dst(pp, b), sem_o[b]).wait()

    def permute(pp, b, bp):
        base = pp * _P
        lane = lax.iota(jnp.int32, 16)
        for g in range(_P // 16):
            rid = lane + 16 * g
            hs = hsel_v[pl.ds(base + 16 * g, 16)]

            def per_s(s, _):
                rot = lax.bitwise_and(lane + s, 15)
                for q in range(_DIM // 16):
                    dl = rot + 16 * q
                    vals = plsc.load_gather(rows_v.at[b], [rid, hs + dl])
                    plsc.store_scatter(pan_v.at[bp], [dl, rid], vals)
                return ()

            lax.fori_loop(0, 16, per_s, (), unroll=8)

    for b in range(4):
        gather(b, b)

    def group(g4, _):
        for b in range(4):
            pp = g4 * 4 + b
            bp = b % 2
            gather_wait(pp, b)

            @pl.when(pp >= 2)
            def _():
                pan_wait(pp - 2, bp)

            permute(pp, b, bp)
            pan_start(pp, bp)

            @pl.when(pp + 4 < _NPAN)
            def _():
                gather(pp + 4, b)
        return ()

    lax.fori_loop(0, _NPAN // 4, group, (), unroll=False)
    pan_wait(_NPAN - 2, 0)
    pan_wait(_NPAN - 1, 1)


@jax.jit
def kernel(x, weight):
    xT = x.T
    mesh = plsc.VectorSubcoreMesh(core_axis_name="c", subcore_axis_name="s")
    w2 = pl.kernel(
        _repack_body,
        out_type=jax.ShapeDtypeStruct((_HALF, 128), jnp.float32),
        mesh=mesh,
        scratch_types=[
            pltpu.VMEM((2, _RB, _DIM), jnp.float32),
            pltpu.VMEM((2, _RB, _DIM), jnp.float32),
            pltpu.VMEM((2, _RB, 128), jnp.float32),
        ] + [pltpu.SemaphoreType.DMA] * 6,
        compiler_params=pltpu.CompilerParams(needs_layout_passes=False),
    )(weight)
    outT = pl.kernel(
        _emb_body,
        out_type=jax.ShapeDtypeStruct((_NJ, _DIM, _NI), jnp.float32),
        mesh=mesh,
        scratch_types=[
            pltpu.VMEM((_NJ, _IPW), jnp.int32),
            pltpu.VMEM((_NJ * _IPW,), jnp.int32),
            pltpu.VMEM((_NJ * _IPW,), jnp.int32),
            pltpu.VMEM((4, _P, 128), jnp.float32),
            pltpu.VMEM((2, _DIM, _P), jnp.float32),
        ] + [pltpu.SemaphoreType.DMA] * 6,
        compiler_params=pltpu.CompilerParams(needs_layout_passes=False),
    )(xT, w2)
    return outT.transpose(2, 0, 1)


# speedup vs baseline: 1.5615x; 1.5615x over previous
import jax
import jax.numpy as jnp
from jax import lax
from jax.experimental import pallas as pl
from jax.experimental.pallas import tpu as pltpu
from jax.experimental.pallas import tpu_sc as plsc

_DIM = 64
_NJ = 20
_NI = 16384
_NC = 2
_NS = 16
_NW = _NC * _NS

_IPW = _NI // _NW
_P = 128
_NPAN = _NJ * (_IPW // _P)


def _emb_body(xT_hbm, w2_hbm, outT_hbm, xT_v, idx_v, hsel_v, rows_v, pan_v,
              sg0, sg1, sg2, sg3, so0, so1):
    sem_g = (sg0, sg1, sg2, sg3)
    sem_o = (so0, so1)
    wid = lax.axis_index("s") * _NC + lax.axis_index("c")
    i0 = wid * _IPW

    pltpu.sync_copy(xT_hbm.at[:, pl.ds(i0, _IPW)], xT_v)

    def prep(g, _):
        j = g // (_IPW // 16)
        t = g % (_IPW // 16)
        v = xT_v[j, pl.ds(t * 16, 16)]
        idx_v[pl.ds(g * 16, 16)] = lax.shift_right_logical(v, 1)
        hsel_v[pl.ds(g * 16, 16)] = lax.shift_left(lax.rem(v, 2), 6)
        return ()

    lax.fori_loop(0, _NJ * (_IPW // 16), prep, (), unroll=False)

    def gather(pp, b):
        pltpu.async_copy(
            w2_hbm.at[idx_v.at[pl.ds(pp * _P, _P)]], rows_v.at[b], sem_g[b])

    def gather_wait(pp, b):
        pltpu.make_async_copy(
            w2_hbm.at[idx_v.at[pl.ds(pp * _P, _P)]], rows_v.at[b],
            sem_g[b]).wait()

    def pan_dst(pp, b):
        j = pp // (_IPW // _P)
        blk = pp % (_IPW // _P)
        return outT_hbm.at[j, :, pl.ds(i0 + blk * _P, _P)]

    def pan_start(pp, b):
        pltpu.async_copy(pan_v.at[b], pan_dst(pp, b), sem_o[b])

    def pan_wait(pp, b):
        pltpu.make_async_copy(pan_v.at[b], pan_dst(pp, b), sem_o[b]).wait()

    def permute(pp, b, bp):
        base = pp * _P
        lane = lax.iota(jnp.int32, 16)
        for g in range(_P // 16):
            rid = lane + 16 * g
            hs = hsel_v[pl.ds(base + 16 * g, 16)]

            def per_s(s, _):
                rot = lax.bitwise_and(lane + s, 15)
                for q in range(_DIM // 16):
                    dl = rot + 16 * q
                    vals = plsc.load_gather(rows_v.at[b], [rid, hs + dl])
                    plsc.store_scatter(pan_v.at[bp], [dl, rid], vals)
                return ()

            lax.fori_loop(0, 16, per_s, (), unroll=8)

    for b in range(4):
        gather(b, b)

    def group(g4, _):
        for b in range(4):
            pp = g4 * 4 + b
            bp = b % 2
            gather_wait(pp, b)

            @pl.when(pp >= 2)
            def _():
                pan_wait(pp - 2, bp)

            permute(pp, b, bp)
            pan_start(pp, bp)

            @pl.when(pp + 4 < _NPAN)
            def _():
                gather(pp + 4, b)
        return ()

    lax.fori_loop(0, _NPAN // 4, group, (), unroll=False)
    pan_wait(_NPAN - 2, 0)
    pan_wait(_NPAN - 1, 1)


@jax.jit
def kernel(x, weight):
    xT = x.T
    w2 = weight.reshape(500000, 128)
    mesh = plsc.VectorSubcoreMesh(core_axis_name="c", subcore_axis_name="s")
    outT = pl.kernel(
        _emb_body,
        out_type=jax.ShapeDtypeStruct((_NJ, _DIM, _NI), jnp.float32),
        mesh=mesh,
        scratch_types=[
            pltpu.VMEM((_NJ, _IPW), jnp.int32),
            pltpu.VMEM((_NJ * _IPW,), jnp.int32),
            pltpu.VMEM((_NJ * _IPW,), jnp.int32),
            pltpu.VMEM((4, _P, 128), jnp.float32),
            pltpu.VMEM((2, _DIM, _P), jnp.float32),
        ] + [pltpu.SemaphoreType.DMA] * 6,
        compiler_params=pltpu.CompilerParams(needs_layout_passes=False),
    )(xT, w2)
    return outT.transpose(2, 0, 1)
